# baseline (speedup 1.0000x reference)
import functools

import jax
import jax.numpy as jnp
from jax import lax
from jax.experimental import pallas as pl
from jax.experimental.pallas import tpu as pltpu
from jax.experimental.pallas import tpu_sc as plsc

NPTS = 10000
NS = 16
CHN = 128
NSH = 16
NPAD = 10240
BQ = 128
BR = 1000
MTOT = float(NPTS * NS)


def _proj_body(x_ref, qwT, kwT, vwT, qb, kb, vb, q_out, k_out, v_out):
    xb = x_ref[...]
    q_out[...] = jnp.dot(xb, qwT[...], preferred_element_type=jnp.float32) + qb[...]
    k_out[...] = jnp.dot(xb, kwT[...], preferred_element_type=jnp.float32) + kb[...]
    v_out[...] = jnp.dot(xb, vwT[...], preferred_element_type=jnp.float32) + vb[...]


def _proj(x, qw, qb, kw, kb, vw, vb):
    n = x.shape[0]
    grid = (n // BR,)
    cspec = pl.BlockSpec((CHN, CHN), lambda i: (0, 0))
    bspec = pl.BlockSpec((1, CHN), lambda i: (0, 0))
    rspec = pl.BlockSpec((BR, CHN), lambda i: (i, 0))
    return pl.pallas_call(
        _proj_body,
        grid=grid,
        in_specs=[rspec, cspec, cspec, cspec, bspec, bspec, bspec],
        out_specs=[rspec, rspec, rspec],
        out_shape=[jax.ShapeDtypeStruct((n, CHN), jnp.float32)] * 3,
    )(x, qw.T, kw.T, vw.T, qb[None], kb[None], vb[None])


_KEEP = 5
_NSL = NPAD // 128
_IMAX = 0x7F000000


def _knn_body(pq_ref, pT_ref, idx_ref):
    pq = pq_ref[...]
    pT = pT_ref[...]
    sqa = jnp.sum(pT * pT, axis=0, keepdims=True)
    sqq = jnp.sum(pq * pq, axis=1, keepdims=True)
    cross = (pq[:, 0:1] * pT[0:1, :] + pq[:, 1:2] * pT[1:2, :]
             + pq[:, 2:3] * pT[2:3, :])
    d = sqq + (sqa - 2.0 * cross)
    col = lax.broadcasted_iota(jnp.int32, d.shape, 1)
    di = jax.lax.bitcast_convert_type(d, jnp.int32)
    di = jnp.where(di >= 0, di, di ^ jnp.int32(0x7FFFFFFF))
    di = di & jnp.int32(~0x7F)
    di = jnp.where(col < NPTS, di, jnp.int32(_IMAX))

    keep = [jnp.full((BQ, 128), _IMAX, jnp.int32)] * _KEEP
    for s in range(_NSL):
        nk = di[:, s * 128:(s + 1) * 128] | jnp.int32(s)
        for r in range(_KEEP):
            swap = nk < keep[r]
            keep[r], nk = (jnp.where(swap, nk, keep[r]),
                           jnp.where(swap, keep[r], nk))

    kd = jnp.concatenate(keep, axis=1)
    lane = lax.broadcasted_iota(jnp.int32, (BQ, 128), 1)
    lanes = jnp.concatenate([lane] * _KEEP, axis=1)
    cols = []
    for _ in range(NS):
        m = jnp.min(kd, axis=1, keepdims=True)
        hit = kd == m
        ids = ((kd & jnp.int32(0x7F)) << 7) | lanes
        ci = jnp.min(jnp.where(hit, ids, jnp.int32(2**30)), axis=1, keepdims=True)
        kd = jnp.where(jnp.logical_and(hit, ids == ci), jnp.int32(_IMAX), kd)
        cols.append(ci)
    idx_ref[...] = jnp.concatenate(cols, axis=1)


def _knn(pp, ppT):
    grid = (NPAD // BQ,)
    return pl.pallas_call(
        _knn_body,
        grid=grid,
        in_specs=[pl.BlockSpec((BQ, 8), lambda i: (i, 0)),
                  pl.BlockSpec((8, NPAD), lambda i: (0, 0))],
        out_specs=pl.BlockSpec((BQ, NS), lambda i: (i, 0)),
        out_shape=jax.ShapeDtypeStruct((NPAD, NS), jnp.int32),
    )(pp, ppT)


_B = NPTS * NS
_NW = 32
_PW = _B // _NW
_NCHK = 25
_CHK = _PW // _NCHK


def _gather3(ktab, vtab, ptab, idx1d):
    mesh = plsc.VectorSubcoreMesh(core_axis_name="c", subcore_axis_name="s")

    @functools.partial(
        pl.kernel, mesh=mesh,
        out_type=[jax.ShapeDtypeStruct((_B, CHN), jnp.float32),
                  jax.ShapeDtypeStruct((_B, CHN), jnp.float32),
                  jax.ShapeDtypeStruct((_B, CHN), jnp.float32)],
        scratch_types=[pltpu.VMEM((_PW,), jnp.int32),
                       pltpu.VMEM((_CHK, CHN), jnp.float32),
                       pltpu.VMEM((_CHK, CHN), jnp.float32),
                       pltpu.VMEM((_CHK, CHN), jnp.float32),
                       pltpu.SemaphoreType.DMA,
                       pltpu.SemaphoreType.DMA,
                       pltpu.SemaphoreType.DMA],
    )
    def gk(kt_h, vt_h, pt_h, idx_h, kg_h, vg_h, pg_h,
           idx_v, kbuf, vbuf, pbuf, s1, s2, s3):
        wid = lax.axis_index("s") * 2 + lax.axis_index("c")
        base = wid * _PW
        pltpu.sync_copy(idx_h.at[pl.ds(base, _PW)], idx_v)

        def body(c, carry):
            cb = base + c * _CHK
            ic = idx_v.at[pl.ds(c * _CHK, _CHK)]
            a1 = pltpu.async_copy(kt_h.at[ic], kbuf, s1)
            a2 = pltpu.async_copy(vt_h.at[ic], vbuf, s2)
            a3 = pltpu.async_copy(pt_h.at[ic], pbuf, s3)
            a1.wait()
            pltpu.sync_copy(kbuf, kg_h.at[pl.ds(cb, _CHK)])
            a2.wait()
            pltpu.sync_copy(vbuf, vg_h.at[pl.ds(cb, _CHK)])
            a3.wait()
            pltpu.sync_copy(pbuf, pg_h.at[pl.ds(cb, _CHK)])
            return carry

        lax.fori_loop(0, _NCHK, body, 0)

    return gk(ktab, vtab, ptab, idx1d)


def _s1_body(pg_ref, pq_ref, w1_ref, b1_ref, y1_ref, acc_ref):
    i, j = pl.program_id(0), pl.program_id(1)
    prel = pg_ref[0] - pq_ref[...]
    y = jnp.dot(prel, w1_ref[...], preferred_element_type=jnp.float32) + b1_ref[...]
    y1_ref[0] = y

    @pl.when(jnp.logical_and(j == 0, i == 0))
    def _():
        acc_ref[...] = jnp.zeros_like(acc_ref)

    acc_ref[0:1, :] += jnp.sum(y, axis=0, keepdims=True)
    acc_ref[1:2, :] += jnp.sum(y * y, axis=0, keepdims=True)


def _s1(pg3, pq, w1p, b1p):
    grid = (NPTS // BR, NS)
    return pl.pallas_call(
        _s1_body,
        grid=grid,
        in_specs=[pl.BlockSpec((1, BR, CHN), lambda i, j: (j, i, 0)),
                  pl.BlockSpec((BR, CHN), lambda i, j: (i, 0)),
                  pl.BlockSpec((CHN, 16), lambda i, j: (0, 0)),
                  pl.BlockSpec((1, 16), lambda i, j: (0, 0))],
        out_specs=[pl.BlockSpec((1, BR, 16), lambda i, j: (j, i, 0)),
                   pl.BlockSpec((8, 16), lambda i, j: (0, 0))],
        out_shape=[jax.ShapeDtypeStruct((NS, NPTS, 16), jnp.float32),
                   jax.ShapeDtypeStruct((8, 16), jnp.float32)],
    )(pg3, pq, w1p, b1p)


def _rqk(y1_blk, kg_blk, xq_blk, g1, bn1, w2p, b2):
    y = jax.nn.relu(y1_blk * g1 + bn1)
    pr2 = jnp.dot(y, w2p, preferred_element_type=jnp.float32) + b2
    rqk = kg_blk - xq_blk + pr2
    return rqk, pr2


def _s2_body(y1_ref, kg_ref, xq_ref, g1_ref, bn1_ref, w2_ref, b2_ref, acc_ref):
    i, j = pl.program_id(0), pl.program_id(1)
    rqk, _ = _rqk(y1_ref[0], kg_ref[0], xq_ref[...],
                  g1_ref[...], bn1_ref[...], w2_ref[...], b2_ref[...])

    @pl.when(jnp.logical_and(j == 0, i == 0))
    def _():
        acc_ref[...] = jnp.zeros_like(acc_ref)

    acc_ref[0:1, :] += jnp.sum(rqk, axis=0, keepdims=True)
    acc_ref[1:2, :] += jnp.sum(rqk * rqk, axis=0, keepdims=True)


def _s2(y1a, kg3, xq, g1, bn1, w2p, b2):
    grid = (NPTS // BR, NS)
    return pl.pallas_call(
        _s2_body,
        grid=grid,
        in_specs=[pl.BlockSpec((1, BR, 16), lambda i, j: (j, i, 0)),
                  pl.BlockSpec((1, BR, CHN), lambda i, j: (j, i, 0)),
                  pl.BlockSpec((BR, CHN), lambda i, j: (i, 0)),
                  pl.BlockSpec((1, 16), lambda i, j: (0, 0)),
                  pl.BlockSpec((1, 16), lambda i, j: (0, 0)),
                  pl.BlockSpec((16, CHN), lambda i, j: (0, 0)),
                  pl.BlockSpec((1, CHN), lambda i, j: (0, 0))],
        out_specs=pl.BlockSpec((8, CHN), lambda i, j: (0, 0)),
        out_shape=jax.ShapeDtypeStruct((8, CHN), jnp.float32),
    )(y1a, kg3, xq, g1, bn1, w2p, b2)


def _w1_body(y1_ref, kg_ref, xq_ref, g1_ref, bn1_ref, w2_ref, b2_ref,
             g2_ref, bn2_ref, ww1_ref, wb1_ref, w1out_ref, acc_ref):
    i, j = pl.program_id(0), pl.program_id(1)
    rqk, _ = _rqk(y1_ref[0], kg_ref[0], xq_ref[...],
                  g1_ref[...], bn1_ref[...], w2_ref[...], b2_ref[...])
    u = jax.nn.relu(rqk * g2_ref[...] + bn2_ref[...])
    w1v = jnp.dot(u, ww1_ref[...], preferred_element_type=jnp.float32) + wb1_ref[...]
    w1out_ref[0] = w1v

    @pl.when(jnp.logical_and(j == 0, i == 0))
    def _():
        acc_ref[...] = jnp.zeros_like(acc_ref)

    acc_ref[0:1, :] += jnp.sum(w1v, axis=0, keepdims=True)
    acc_ref[1:2, :] += jnp.sum(w1v * w1v, axis=0, keepdims=True)


def _w1(y1a, kg3, xq, g1, bn1, w2p, b2, g2, bn2, ww1T, wb1):
    grid = (NPTS // BR, NS)
    return pl.pallas_call(
        _w1_body,
        grid=grid,
        in_specs=[pl.BlockSpec((1, BR, 16), lambda i, j: (j, i, 0)),
                  pl.BlockSpec((1, BR, CHN), lambda i, j: (j, i, 0)),
                  pl.BlockSpec((BR, CHN), lambda i, j: (i, 0)),
                  pl.BlockSpec((1, 16), lambda i, j: (0, 0)),
                  pl.BlockSpec((1, 16), lambda i, j: (0, 0)),
                  pl.BlockSpec((16, CHN), lambda i, j: (0, 0)),
                  pl.BlockSpec((1, CHN), lambda i, j: (0, 0)),
                  pl.BlockSpec((1, CHN), lambda i, j: (0, 0)),
                  pl.BlockSpec((1, CHN), lambda i, j: (0, 0)),
                  pl.BlockSpec((CHN, NSH), lambda i, j: (0, 0)),
                  pl.BlockSpec((1, NSH), lambda i, j: (0, 0))],
        out_specs=[pl.BlockSpec((1, BR, NSH), lambda i, j: (j, i, 0)),
                   pl.BlockSpec((8, NSH), lambda i, j: (0, 0))],
        out_shape=[jax.ShapeDtypeStruct((NS, NPTS, NSH), jnp.float32),
                   jax.ShapeDtypeStruct((8, NSH), jnp.float32)],
    )(y1a, kg3, xq, g1, bn1, w2p, b2, g2, bn2, ww1T, wb1)


def _fin_body(w1_ref, vg_ref, y1_ref, g1_ref, bn1_ref, w2p_ref, b2_ref,
              g3_ref, bn3_ref, ww2_ref, wb2_ref, out_ref):
    g3 = g3_ref[...]
    bn3 = bn3_ref[...]
    ww2 = ww2_ref[...]
    wb2 = wb2_ref[...]
    zs = []
    for j in range(NS):
        u = jax.nn.relu(w1_ref[j] * g3 + bn3)
        zs.append(jnp.dot(u, ww2, preferred_element_type=jnp.float32) + wb2)
    m = zs[0]
    for j in range(1, NS):
        m = jnp.maximum(m, zs[j])
    es = [jnp.exp(z - m) for z in zs]
    tot = es[0]
    for j in range(1, NS):
        tot = tot + es[j]
    inv = 1.0 / tot
    acc = jnp.zeros_like(out_ref)
    for j in range(NS):
        y = jax.nn.relu(y1_ref[j] * g1_ref[...] + bn1_ref[...])
        pr2 = jnp.dot(y, w2p_ref[...], preferred_element_type=jnp.float32) + b2_ref[...]
        v = vg_ref[j] + pr2
        wj = es[j] * inv
        wt = jnp.concatenate([wj] * 8, axis=1)
        acc = acc + v * wt
    out_ref[...] = acc


BRF = 400


def _final(w1a, vg3, y1a, g1, bn1, w2p, b2, g3, bn3, ww2T, wb2):
    grid = (NPTS // BRF,)
    return pl.pallas_call(
        _fin_body,
        grid=grid,
        in_specs=[pl.BlockSpec((NS, BRF, NSH), lambda i: (0, i, 0)),
                  pl.BlockSpec((NS, BRF, CHN), lambda i: (0, i, 0)),
                  pl.BlockSpec((NS, BRF, 16), lambda i: (0, i, 0)),
                  pl.BlockSpec((1, 16), lambda i: (0, 0)),
                  pl.BlockSpec((1, 16), lambda i: (0, 0)),
                  pl.BlockSpec((16, CHN), lambda i: (0, 0)),
                  pl.BlockSpec((1, CHN), lambda i: (0, 0)),
                  pl.BlockSpec((1, NSH), lambda i: (0, 0)),
                  pl.BlockSpec((1, NSH), lambda i: (0, 0)),
                  pl.BlockSpec((NSH, NSH), lambda i: (0, 0)),
                  pl.BlockSpec((1, NSH), lambda i: (0, 0))],
        out_specs=pl.BlockSpec((BRF, CHN), lambda i: (i, 0)),
        out_shape=jax.ShapeDtypeStruct((NPTS, CHN), jnp.float32),
    )(w1a, vg3, y1a, g1, bn1, w2p, b2, g3, bn3, ww2T, wb2)


def kernel(p, x, o, qw, qb, kw, kb, vw, vb, pw1, pb1, pg, pbeta, pw2, pb2,
           wg1, wbeta1, ww1, wlb1, wg2, wbeta2, ww2, wlb2):
    eps = 1e-5
    xq, xk, xv = _proj(x, qw, qb, kw, kb, vw, vb)

    pp = jnp.zeros((NPAD, 8), jnp.float32).at[:NPTS, :3].set(p)
    idx = _knn(pp, pp.T)[:NPTS]

    idx_t = idx.T.reshape(-1)
    pq16 = jnp.zeros((NPTS, CHN), jnp.float32).at[:, :3].set(p)
    kg, vg, pg_rows = _gather3(xk, xv, pq16, idx_t)
    kg3 = kg.reshape(NS, NPTS, CHN)
    vg3 = vg.reshape(NS, NPTS, CHN)
    pg3 = pg_rows.reshape(NS, NPTS, CHN)

    w1p = jnp.zeros((CHN, 16), jnp.float32).at[:3, :3].set(pw1.T)
    b1p = jnp.zeros((1, 16), jnp.float32).at[0, :3].set(pb1)
    w2p = jnp.zeros((16, CHN), jnp.float32).at[:3, :].set(pw2.T)
    b2 = pb2[None]
    pg_p = jnp.zeros((16,), jnp.float32).at[:3].set(pg)
    pbeta_p = jnp.zeros((16,), jnp.float32).at[:3].set(pbeta)

    y1a, s1 = _s1(pg3, pq16, w1p, b1p)
    m1 = s1[0] / MTOT
    v1 = s1[1] / MTOT - m1 * m1
    g1v = pg_p / jnp.sqrt(v1 + eps)
    g1 = g1v[None]
    bn1 = (pbeta_p - m1 * g1v)[None]

    s2 = _s2(y1a, kg3, xq, g1, bn1, w2p, b2)
    m2 = s2[0] / MTOT
    v2 = s2[1] / MTOT - m2 * m2
    g2v = wg1 / jnp.sqrt(v2 + eps)
    g2 = g2v[None]
    bn2 = (wbeta1 - m2 * g2v)[None]

    w1a, s3 = _w1(y1a, kg3, xq, g1, bn1, w2p, b2, g2, bn2,
                  ww1.T, wlb1[None])
    m3 = s3[0] / MTOT
    v3 = s3[1] / MTOT - m3 * m3
    g3v = wg2 / jnp.sqrt(v3 + eps)
    g3 = g3v[None]
    bn3 = (wbeta2 - m3 * g3v)[None]

    return _final(w1a, vg3, y1a, g1, bn1, w2p, b2,
                  g3, bn3, ww2.T, wlb2[None])


# speedup vs baseline: 1.1984x; 1.1984x over previous
import functools

import jax
import jax.numpy as jnp
from jax import lax
from jax.experimental import pallas as pl
from jax.experimental.pallas import tpu as pltpu
from jax.experimental.pallas import tpu_sc as plsc

NPTS = 10000
NS = 16
CHN = 128
NSH = 16
NPAD = 10240
BQ = 256
BR = 1000
MTOT = float(NPTS * NS)


def _proj_body(x_ref, qwT, kwT, vwT, qb, kb, vb, q_out, k_out, v_out):
    xb = x_ref[...]
    q_out[...] = jnp.dot(xb, qwT[...], preferred_element_type=jnp.float32) + qb[...]
    k_out[...] = jnp.dot(xb, kwT[...], preferred_element_type=jnp.float32) + kb[...]
    v_out[...] = jnp.dot(xb, vwT[...], preferred_element_type=jnp.float32) + vb[...]


def _proj(x, qw, qb, kw, kb, vw, vb):
    n = x.shape[0]
    grid = (n // BR,)
    cspec = pl.BlockSpec((CHN, CHN), lambda i: (0, 0))
    bspec = pl.BlockSpec((1, CHN), lambda i: (0, 0))
    rspec = pl.BlockSpec((BR, CHN), lambda i: (i, 0))
    return pl.pallas_call(
        _proj_body,
        grid=grid,
        in_specs=[rspec, cspec, cspec, cspec, bspec, bspec, bspec],
        out_specs=[rspec, rspec, rspec],
        out_shape=[jax.ShapeDtypeStruct((n, CHN), jnp.float32)] * 3,
    )(x, qw.T, kw.T, vw.T, qb[None], kb[None], vb[None])


_KEEP = 5
_NSL = NPAD // 128
_IMAX = 0x7F000000


def _knn_body(pq_ref, pT_ref, idx_ref):
    pq = pq_ref[...]
    pT = pT_ref[...]
    cross = lax.dot_general(pq, pT, (((1,), (0,)), ((), ())),
                            preferred_element_type=jnp.float32)
    sqa = jnp.sum(pT * pT, axis=0, keepdims=True)
    sqq = jnp.sum(pq * pq, axis=1, keepdims=True)
    d = sqq + (sqa - 2.0 * cross)
    col = lax.broadcasted_iota(jnp.int32, d.shape, 1)
    di = jax.lax.bitcast_convert_type(d, jnp.int32)
    di = jnp.where(di >= 0, di, di ^ jnp.int32(0x7FFFFFFF))
    di = di & jnp.int32(~0x7F)
    di = jnp.where(col < NPTS, di, jnp.int32(_IMAX))

    keep = [jnp.full((BQ, 128), _IMAX, jnp.int32)] * _KEEP
    for s in range(_NSL):
        nk = di[:, s * 128:(s + 1) * 128] | jnp.int32(s)
        for r in range(_KEEP):
            swap = nk < keep[r]
            keep[r], nk = (jnp.where(swap, nk, keep[r]),
                           jnp.where(swap, keep[r], nk))

    kd = jnp.concatenate(keep, axis=1)
    lane = lax.broadcasted_iota(jnp.int32, (BQ, 128), 1)
    lanes = jnp.concatenate([lane] * _KEEP, axis=1)
    cols = []
    for _ in range(NS):
        m = jnp.min(kd, axis=1, keepdims=True)
        hit = kd == m
        ids = ((kd & jnp.int32(0x7F)) << 7) | lanes
        ci = jnp.min(jnp.where(hit, ids, jnp.int32(2**30)), axis=1, keepdims=True)
        kd = jnp.where(jnp.logical_and(hit, ids == ci), jnp.int32(_IMAX), kd)
        cols.append(ci)
    idx_ref[...] = jnp.concatenate(cols, axis=1)


def _knn(pp, ppT):
    grid = (NPAD // BQ,)
    return pl.pallas_call(
        _knn_body,
        grid=grid,
        in_specs=[pl.BlockSpec((BQ, 8), lambda i: (i, 0)),
                  pl.BlockSpec((8, NPAD), lambda i: (0, 0))],
        out_specs=pl.BlockSpec((BQ, NS), lambda i: (i, 0)),
        out_shape=jax.ShapeDtypeStruct((NPAD, NS), jnp.int32),
    )(pp, ppT)


_B = NPTS * NS
_NW = 32
_PW = _B // _NW
_NCHK = 25
_CHK = _PW // _NCHK


def _gather3(ktab, vtab, ptab, idx1d):
    mesh = plsc.VectorSubcoreMesh(core_axis_name="c", subcore_axis_name="s")

    @functools.partial(
        pl.kernel, mesh=mesh,
        out_type=[jax.ShapeDtypeStruct((_B, CHN), jnp.float32),
                  jax.ShapeDtypeStruct((_B, CHN), jnp.float32),
                  jax.ShapeDtypeStruct((_B, CHN), jnp.float32)],
        scratch_types=[pltpu.VMEM((_PW,), jnp.int32),
                       pltpu.VMEM((_CHK, CHN), jnp.float32),
                       pltpu.VMEM((_CHK, CHN), jnp.float32),
                       pltpu.VMEM((_CHK, CHN), jnp.float32),
                       pltpu.SemaphoreType.DMA,
                       pltpu.SemaphoreType.DMA,
                       pltpu.SemaphoreType.DMA],
    )
    def gk(kt_h, vt_h, pt_h, idx_h, kg_h, vg_h, pg_h,
           idx_v, kbuf, vbuf, pbuf, s1, s2, s3):
        wid = lax.axis_index("s") * 2 + lax.axis_index("c")
        base = wid * _PW
        pltpu.sync_copy(idx_h.at[pl.ds(base, _PW)], idx_v)

        def body(c, carry):
            cb = base + c * _CHK
            ic = idx_v.at[pl.ds(c * _CHK, _CHK)]
            a1 = pltpu.async_copy(kt_h.at[ic], kbuf, s1)
            a2 = pltpu.async_copy(vt_h.at[ic], vbuf, s2)
            a3 = pltpu.async_copy(pt_h.at[ic], pbuf, s3)
            a1.wait()
            pltpu.sync_copy(kbuf, kg_h.at[pl.ds(cb, _CHK)])
            a2.wait()
            pltpu.sync_copy(vbuf, vg_h.at[pl.ds(cb, _CHK)])
            a3.wait()
            pltpu.sync_copy(pbuf, pg_h.at[pl.ds(cb, _CHK)])
            return carry

        lax.fori_loop(0, _NCHK, body, 0)

    return gk(ktab, vtab, ptab, idx1d)


def _s1_body(pg_ref, pq_ref, w1_ref, b1_ref, y1_ref, acc_ref):
    i, j = pl.program_id(0), pl.program_id(1)
    prel = pg_ref[0] - pq_ref[...]
    y = jnp.dot(prel, w1_ref[...], preferred_element_type=jnp.float32) + b1_ref[...]
    y1_ref[0] = y

    @pl.when(jnp.logical_and(j == 0, i == 0))
    def _():
        acc_ref[...] = jnp.zeros_like(acc_ref)

    acc_ref[0:1, :] += jnp.sum(y, axis=0, keepdims=True)
    acc_ref[1:2, :] += jnp.sum(y * y, axis=0, keepdims=True)


def _s1(pg3, pq, w1p, b1p):
    grid = (NPTS // BR, NS)
    return pl.pallas_call(
        _s1_body,
        grid=grid,
        in_specs=[pl.BlockSpec((1, BR, CHN), lambda i, j: (j, i, 0)),
                  pl.BlockSpec((BR, CHN), lambda i, j: (i, 0)),
                  pl.BlockSpec((CHN, 16), lambda i, j: (0, 0)),
                  pl.BlockSpec((1, 16), lambda i, j: (0, 0))],
        out_specs=[pl.BlockSpec((1, BR, 16), lambda i, j: (j, i, 0)),
                   pl.BlockSpec((8, 16), lambda i, j: (0, 0))],
        out_shape=[jax.ShapeDtypeStruct((NS, NPTS, 16), jnp.float32),
                   jax.ShapeDtypeStruct((8, 16), jnp.float32)],
    )(pg3, pq, w1p, b1p)


def _rqk(y1_blk, kg_blk, xq_blk, g1, bn1, w2p, b2):
    y = jax.nn.relu(y1_blk * g1 + bn1)
    pr2 = jnp.dot(y, w2p, preferred_element_type=jnp.float32) + b2
    rqk = kg_blk - xq_blk + pr2
    return rqk, pr2


def _s2_body(y1_ref, kg_ref, xq_ref, g1_ref, bn1_ref, w2_ref, b2_ref, acc_ref):
    i, j = pl.program_id(0), pl.program_id(1)
    rqk, _ = _rqk(y1_ref[0], kg_ref[0], xq_ref[...],
                  g1_ref[...], bn1_ref[...], w2_ref[...], b2_ref[...])

    @pl.when(jnp.logical_and(j == 0, i == 0))
    def _():
        acc_ref[...] = jnp.zeros_like(acc_ref)

    acc_ref[0:1, :] += jnp.sum(rqk, axis=0, keepdims=True)
    acc_ref[1:2, :] += jnp.sum(rqk * rqk, axis=0, keepdims=True)


def _s2(y1a, kg3, xq, g1, bn1, w2p, b2):
    grid = (NPTS // BR, NS)
    return pl.pallas_call(
        _s2_body,
        grid=grid,
        in_specs=[pl.BlockSpec((1, BR, 16), lambda i, j: (j, i, 0)),
                  pl.BlockSpec((1, BR, CHN), lambda i, j: (j, i, 0)),
                  pl.BlockSpec((BR, CHN), lambda i, j: (i, 0)),
                  pl.BlockSpec((1, 16), lambda i, j: (0, 0)),
                  pl.BlockSpec((1, 16), lambda i, j: (0, 0)),
                  pl.BlockSpec((16, CHN), lambda i, j: (0, 0)),
                  pl.BlockSpec((1, CHN), lambda i, j: (0, 0))],
        out_specs=pl.BlockSpec((8, CHN), lambda i, j: (0, 0)),
        out_shape=jax.ShapeDtypeStruct((8, CHN), jnp.float32),
    )(y1a, kg3, xq, g1, bn1, w2p, b2)


def _w1_body(y1_ref, kg_ref, xq_ref, g1_ref, bn1_ref, w2_ref, b2_ref,
             g2_ref, bn2_ref, ww1_ref, wb1_ref, w1out_ref, acc_ref):
    i, j = pl.program_id(0), pl.program_id(1)
    rqk, _ = _rqk(y1_ref[0], kg_ref[0], xq_ref[...],
                  g1_ref[...], bn1_ref[...], w2_ref[...], b2_ref[...])
    u = jax.nn.relu(rqk * g2_ref[...] + bn2_ref[...])
    w1v = jnp.dot(u, ww1_ref[...], preferred_element_type=jnp.float32) + wb1_ref[...]
    w1out_ref[0] = w1v

    @pl.when(jnp.logical_and(j == 0, i == 0))
    def _():
        acc_ref[...] = jnp.zeros_like(acc_ref)

    acc_ref[0:1, :] += jnp.sum(w1v, axis=0, keepdims=True)
    acc_ref[1:2, :] += jnp.sum(w1v * w1v, axis=0, keepdims=True)


def _w1(y1a, kg3, xq, g1, bn1, w2p, b2, g2, bn2, ww1T, wb1):
    grid = (NPTS // BR, NS)
    return pl.pallas_call(
        _w1_body,
        grid=grid,
        in_specs=[pl.BlockSpec((1, BR, 16), lambda i, j: (j, i, 0)),
                  pl.BlockSpec((1, BR, CHN), lambda i, j: (j, i, 0)),
                  pl.BlockSpec((BR, CHN), lambda i, j: (i, 0)),
                  pl.BlockSpec((1, 16), lambda i, j: (0, 0)),
                  pl.BlockSpec((1, 16), lambda i, j: (0, 0)),
                  pl.BlockSpec((16, CHN), lambda i, j: (0, 0)),
                  pl.BlockSpec((1, CHN), lambda i, j: (0, 0)),
                  pl.BlockSpec((1, CHN), lambda i, j: (0, 0)),
                  pl.BlockSpec((1, CHN), lambda i, j: (0, 0)),
                  pl.BlockSpec((CHN, NSH), lambda i, j: (0, 0)),
                  pl.BlockSpec((1, NSH), lambda i, j: (0, 0))],
        out_specs=[pl.BlockSpec((1, BR, NSH), lambda i, j: (j, i, 0)),
                   pl.BlockSpec((8, NSH), lambda i, j: (0, 0))],
        out_shape=[jax.ShapeDtypeStruct((NS, NPTS, NSH), jnp.float32),
                   jax.ShapeDtypeStruct((8, NSH), jnp.float32)],
    )(y1a, kg3, xq, g1, bn1, w2p, b2, g2, bn2, ww1T, wb1)


def _fin_body(w1_ref, vg_ref, y1_ref, g1_ref, bn1_ref, w2p_ref, b2_ref,
              g3_ref, bn3_ref, ww2_ref, wb2_ref, out_ref):
    g3 = g3_ref[...]
    bn3 = bn3_ref[...]
    ww2 = ww2_ref[...]
    wb2 = wb2_ref[...]
    zs = []
    for j in range(NS):
        u = jax.nn.relu(w1_ref[j] * g3 + bn3)
        zs.append(jnp.dot(u, ww2, preferred_element_type=jnp.float32) + wb2)
    m = zs[0]
    for j in range(1, NS):
        m = jnp.maximum(m, zs[j])
    es = [jnp.exp(z - m) for z in zs]
    tot = es[0]
    for j in range(1, NS):
        tot = tot + es[j]
    inv = 1.0 / tot
    acc = jnp.zeros_like(out_ref)
    for j in range(NS):
        y = jax.nn.relu(y1_ref[j] * g1_ref[...] + bn1_ref[...])
        pr2 = jnp.dot(y, w2p_ref[...], preferred_element_type=jnp.float32) + b2_ref[...]
        v = vg_ref[j] + pr2
        wj = es[j] * inv
        wt = jnp.concatenate([wj] * 8, axis=1)
        acc = acc + v * wt
    out_ref[...] = acc


BRF = 400


def _final(w1a, vg3, y1a, g1, bn1, w2p, b2, g3, bn3, ww2T, wb2):
    grid = (NPTS // BRF,)
    return pl.pallas_call(
        _fin_body,
        grid=grid,
        in_specs=[pl.BlockSpec((NS, BRF, NSH), lambda i: (0, i, 0)),
                  pl.BlockSpec((NS, BRF, CHN), lambda i: (0, i, 0)),
                  pl.BlockSpec((NS, BRF, 16), lambda i: (0, i, 0)),
                  pl.BlockSpec((1, 16), lambda i: (0, 0)),
                  pl.BlockSpec((1, 16), lambda i: (0, 0)),
                  pl.BlockSpec((16, CHN), lambda i: (0, 0)),
                  pl.BlockSpec((1, CHN), lambda i: (0, 0)),
                  pl.BlockSpec((1, NSH), lambda i: (0, 0)),
                  pl.BlockSpec((1, NSH), lambda i: (0, 0)),
                  pl.BlockSpec((NSH, NSH), lambda i: (0, 0)),
                  pl.BlockSpec((1, NSH), lambda i: (0, 0))],
        out_specs=pl.BlockSpec((BRF, CHN), lambda i: (i, 0)),
        out_shape=jax.ShapeDtypeStruct((NPTS, CHN), jnp.float32),
    )(w1a, vg3, y1a, g1, bn1, w2p, b2, g3, bn3, ww2T, wb2)


def kernel(p, x, o, qw, qb, kw, kb, vw, vb, pw1, pb1, pg, pbeta, pw2, pb2,
           wg1, wbeta1, ww1, wlb1, wg2, wbeta2, ww2, wlb2):
    eps = 1e-5
    xq, xk, xv = _proj(x, qw, qb, kw, kb, vw, vb)

    pp = jnp.zeros((NPAD, 8), jnp.float32).at[:NPTS, :3].set(p)
    idx = _knn(pp, pp.T)[:NPTS]

    idx_t = idx.T.reshape(-1)
    pq16 = jnp.zeros((NPTS, CHN), jnp.float32).at[:, :3].set(p)
    kg, vg, pg_rows = _gather3(xk, xv, pq16, idx_t)
    kg3 = kg.reshape(NS, NPTS, CHN)
    vg3 = vg.reshape(NS, NPTS, CHN)
    pg3 = pg_rows.reshape(NS, NPTS, CHN)

    w1p = jnp.zeros((CHN, 16), jnp.float32).at[:3, :3].set(pw1.T)
    b1p = jnp.zeros((1, 16), jnp.float32).at[0, :3].set(pb1)
    w2p = jnp.zeros((16, CHN), jnp.float32).at[:3, :].set(pw2.T)
    b2 = pb2[None]
    pg_p = jnp.zeros((16,), jnp.float32).at[:3].set(pg)
    pbeta_p = jnp.zeros((16,), jnp.float32).at[:3].set(pbeta)

    y1a, s1 = _s1(pg3, pq16, w1p, b1p)
    m1 = s1[0] / MTOT
    v1 = s1[1] / MTOT - m1 * m1
    g1v = pg_p / jnp.sqrt(v1 + eps)
    g1 = g1v[None]
    bn1 = (pbeta_p - m1 * g1v)[None]

    s2 = _s2(y1a, kg3, xq, g1, bn1, w2p, b2)
    m2 = s2[0] / MTOT
    v2 = s2[1] / MTOT - m2 * m2
    g2v = wg1 / jnp.sqrt(v2 + eps)
    g2 = g2v[None]
    bn2 = (wbeta1 - m2 * g2v)[None]

    w1a, s3 = _w1(y1a, kg3, xq, g1, bn1, w2p, b2, g2, bn2,
                  ww1.T, wlb1[None])
    m3 = s3[0] / MTOT
    v3 = s3[1] / MTOT - m3 * m3
    g3v = wg2 / jnp.sqrt(v3 + eps)
    g3 = g3v[None]
    bn3 = (wbeta2 - m3 * g3v)[None]

    return _final(w1a, vg3, y1a, g1, bn1, w2p, b2,
                  g3, bn3, ww2.T, wlb2[None])


# speedup vs baseline: 1.2344x; 1.0301x over previous
import functools

import jax
import jax.numpy as jnp
from jax import lax
from jax.experimental import pallas as pl
from jax.experimental.pallas import tpu as pltpu
from jax.experimental.pallas import tpu_sc as plsc

NPTS = 10000
NS = 16
CHN = 128
NSH = 16
NPAD = 10240
BQ = 512
BR = 1000
MTOT = float(NPTS * NS)


def _proj_body(x_ref, qwT, kwT, vwT, qb, kb, vb, q_out, k_out, v_out):
    xb = x_ref[...]
    q_out[...] = jnp.dot(xb, qwT[...], preferred_element_type=jnp.float32) + qb[...]
    k_out[...] = jnp.dot(xb, kwT[...], preferred_element_type=jnp.float32) + kb[...]
    v_out[...] = jnp.dot(xb, vwT[...], preferred_element_type=jnp.float32) + vb[...]


def _proj(x, qw, qb, kw, kb, vw, vb):
    n = x.shape[0]
    grid = (n // BR,)
    cspec = pl.BlockSpec((CHN, CHN), lambda i: (0, 0))
    bspec = pl.BlockSpec((1, CHN), lambda i: (0, 0))
    rspec = pl.BlockSpec((BR, CHN), lambda i: (i, 0))
    return pl.pallas_call(
        _proj_body,
        grid=grid,
        in_specs=[rspec, cspec, cspec, cspec, bspec, bspec, bspec],
        out_specs=[rspec, rspec, rspec],
        out_shape=[jax.ShapeDtypeStruct((n, CHN), jnp.float32)] * 3,
    )(x, qw.T, kw.T, vw.T, qb[None], kb[None], vb[None])


_KEEP = 5
_NSL = NPAD // 128
_IMAX = 0x7F000000


def _knn_body(pq_ref, pT_ref, idx_ref):
    pq = pq_ref[...]
    pT = pT_ref[...]
    cross = lax.dot_general(pq, pT, (((1,), (0,)), ((), ())),
                            preferred_element_type=jnp.float32)
    sqa = jnp.sum(pT * pT, axis=0, keepdims=True)
    sqq = jnp.sum(pq * pq, axis=1, keepdims=True)
    d = sqq + (sqa - 2.0 * cross)
    col = lax.broadcasted_iota(jnp.int32, d.shape, 1)
    di = jax.lax.bitcast_convert_type(d, jnp.int32)
    di = jnp.where(di >= 0, di, di ^ jnp.int32(0x7FFFFFFF))
    di = di & jnp.int32(~0x7F)
    di = jnp.where(col < NPTS, di, jnp.int32(_IMAX))

    keep = [jnp.full((BQ, 128), _IMAX, jnp.int32)] * _KEEP
    for s in range(_NSL):
        nk = di[:, s * 128:(s + 1) * 128] | jnp.int32(s)
        for r in range(_KEEP):
            swap = nk < keep[r]
            keep[r], nk = (jnp.where(swap, nk, keep[r]),
                           jnp.where(swap, keep[r], nk))

    kd = jnp.concatenate(keep, axis=1)
    lane = lax.broadcasted_iota(jnp.int32, (BQ, 128), 1)
    lanes = jnp.concatenate([lane] * _KEEP, axis=1)
    cols = []
    for _ in range(NS):
        m = jnp.min(kd, axis=1, keepdims=True)
        hit = kd == m
        ids = ((kd & jnp.int32(0x7F)) << 7) | lanes
        ci = jnp.min(jnp.where(hit, ids, jnp.int32(2**30)), axis=1, keepdims=True)
        kd = jnp.where(jnp.logical_and(hit, ids == ci), jnp.int32(_IMAX), kd)
        cols.append(ci)
    idx_ref[...] = jnp.concatenate(cols, axis=1)


def _knn(pp, ppT):
    grid = (NPAD // BQ,)
    return pl.pallas_call(
        _knn_body,
        grid=grid,
        in_specs=[pl.BlockSpec((BQ, 8), lambda i: (i, 0)),
                  pl.BlockSpec((8, NPAD), lambda i: (0, 0))],
        out_specs=pl.BlockSpec((BQ, NS), lambda i: (i, 0)),
        out_shape=jax.ShapeDtypeStruct((NPAD, NS), jnp.int32),
    )(pp, ppT)


_B = NPTS * NS
_NW = 32
_PW = _B // _NW
_NCHK = 25
_CHK = _PW // _NCHK


def _gather3(ktab, vtab, ptab, idx1d):
    mesh = plsc.VectorSubcoreMesh(core_axis_name="c", subcore_axis_name="s")

    @functools.partial(
        pl.kernel, mesh=mesh,
        out_type=[jax.ShapeDtypeStruct((_B, CHN), jnp.float32),
                  jax.ShapeDtypeStruct((_B, CHN), jnp.float32),
                  jax.ShapeDtypeStruct((_B, CHN), jnp.float32)],
        scratch_types=[pltpu.VMEM((_PW,), jnp.int32),
                       pltpu.VMEM((_CHK, CHN), jnp.float32),
                       pltpu.VMEM((_CHK, CHN), jnp.float32),
                       pltpu.VMEM((_CHK, CHN), jnp.float32),
                       pltpu.SemaphoreType.DMA,
                       pltpu.SemaphoreType.DMA,
                       pltpu.SemaphoreType.DMA],
    )
    def gk(kt_h, vt_h, pt_h, idx_h, kg_h, vg_h, pg_h,
           idx_v, kbuf, vbuf, pbuf, s1, s2, s3):
        wid = lax.axis_index("s") * 2 + lax.axis_index("c")
        base = wid * _PW
        pltpu.sync_copy(idx_h.at[pl.ds(base, _PW)], idx_v)

        def body(c, carry):
            cb = base + c * _CHK
            ic = idx_v.at[pl.ds(c * _CHK, _CHK)]
            a1 = pltpu.async_copy(kt_h.at[ic], kbuf, s1)
            a2 = pltpu.async_copy(vt_h.at[ic], vbuf, s2)
            a3 = pltpu.async_copy(pt_h.at[ic], pbuf, s3)
            a1.wait()
            pltpu.sync_copy(kbuf, kg_h.at[pl.ds(cb, _CHK)])
            a2.wait()
            pltpu.sync_copy(vbuf, vg_h.at[pl.ds(cb, _CHK)])
            a3.wait()
            pltpu.sync_copy(pbuf, pg_h.at[pl.ds(cb, _CHK)])
            return carry

        lax.fori_loop(0, _NCHK, body, 0)

    return gk(ktab, vtab, ptab, idx1d)


def _s1_body(pg_ref, pq_ref, w1_ref, b1_ref, y1_ref, acc_ref):
    i, j = pl.program_id(0), pl.program_id(1)
    prel = pg_ref[0] - pq_ref[...]
    y = jnp.dot(prel, w1_ref[...], preferred_element_type=jnp.float32) + b1_ref[...]
    y1_ref[0] = y

    @pl.when(jnp.logical_and(j == 0, i == 0))
    def _():
        acc_ref[...] = jnp.zeros_like(acc_ref)

    acc_ref[0:1, :] += jnp.sum(y, axis=0, keepdims=True)
    acc_ref[1:2, :] += jnp.sum(y * y, axis=0, keepdims=True)


def _s1(pg3, pq, w1p, b1p):
    grid = (NPTS // BR, NS)
    return pl.pallas_call(
        _s1_body,
        grid=grid,
        in_specs=[pl.BlockSpec((1, BR, CHN), lambda i, j: (j, i, 0)),
                  pl.BlockSpec((BR, CHN), lambda i, j: (i, 0)),
                  pl.BlockSpec((CHN, 16), lambda i, j: (0, 0)),
                  pl.BlockSpec((1, 16), lambda i, j: (0, 0))],
        out_specs=[pl.BlockSpec((1, BR, 16), lambda i, j: (j, i, 0)),
                   pl.BlockSpec((8, 16), lambda i, j: (0, 0))],
        out_shape=[jax.ShapeDtypeStruct((NS, NPTS, 16), jnp.float32),
                   jax.ShapeDtypeStruct((8, 16), jnp.float32)],
    )(pg3, pq, w1p, b1p)


def _rqk(y1_blk, kg_blk, xq_blk, g1, bn1, w2p, b2):
    y = jax.nn.relu(y1_blk * g1 + bn1)
    pr2 = jnp.dot(y, w2p, preferred_element_type=jnp.float32) + b2
    rqk = kg_blk - xq_blk + pr2
    return rqk, pr2


def _s2_body(y1_ref, kg_ref, xq_ref, g1_ref, bn1_ref, w2_ref, b2_ref, acc_ref):
    i, j = pl.program_id(0), pl.program_id(1)
    rqk, _ = _rqk(y1_ref[0], kg_ref[0], xq_ref[...],
                  g1_ref[...], bn1_ref[...], w2_ref[...], b2_ref[...])

    @pl.when(jnp.logical_and(j == 0, i == 0))
    def _():
        acc_ref[...] = jnp.zeros_like(acc_ref)

    acc_ref[0:1, :] += jnp.sum(rqk, axis=0, keepdims=True)
    acc_ref[1:2, :] += jnp.sum(rqk * rqk, axis=0, keepdims=True)


def _s2(y1a, kg3, xq, g1, bn1, w2p, b2):
    grid = (NPTS // BR, NS)
    return pl.pallas_call(
        _s2_body,
        grid=grid,
        in_specs=[pl.BlockSpec((1, BR, 16), lambda i, j: (j, i, 0)),
                  pl.BlockSpec((1, BR, CHN), lambda i, j: (j, i, 0)),
                  pl.BlockSpec((BR, CHN), lambda i, j: (i, 0)),
                  pl.BlockSpec((1, 16), lambda i, j: (0, 0)),
                  pl.BlockSpec((1, 16), lambda i, j: (0, 0)),
                  pl.BlockSpec((16, CHN), lambda i, j: (0, 0)),
                  pl.BlockSpec((1, CHN), lambda i, j: (0, 0))],
        out_specs=pl.BlockSpec((8, CHN), lambda i, j: (0, 0)),
        out_shape=jax.ShapeDtypeStruct((8, CHN), jnp.float32),
    )(y1a, kg3, xq, g1, bn1, w2p, b2)


def _w1_body(y1_ref, kg_ref, xq_ref, g1_ref, bn1_ref, w2_ref, b2_ref,
             g2_ref, bn2_ref, ww1_ref, wb1_ref, w1out_ref, acc_ref):
    i, j = pl.program_id(0), pl.program_id(1)
    rqk, _ = _rqk(y1_ref[0], kg_ref[0], xq_ref[...],
                  g1_ref[...], bn1_ref[...], w2_ref[...], b2_ref[...])
    u = jax.nn.relu(rqk * g2_ref[...] + bn2_ref[...])
    w1v = jnp.dot(u, ww1_ref[...], preferred_element_type=jnp.float32) + wb1_ref[...]
    w1out_ref[0] = w1v

    @pl.when(jnp.logical_and(j == 0, i == 0))
    def _():
        acc_ref[...] = jnp.zeros_like(acc_ref)

    acc_ref[0:1, :] += jnp.sum(w1v, axis=0, keepdims=True)
    acc_ref[1:2, :] += jnp.sum(w1v * w1v, axis=0, keepdims=True)


def _w1(y1a, kg3, xq, g1, bn1, w2p, b2, g2, bn2, ww1T, wb1):
    grid = (NPTS // BR, NS)
    return pl.pallas_call(
        _w1_body,
        grid=grid,
        in_specs=[pl.BlockSpec((1, BR, 16), lambda i, j: (j, i, 0)),
                  pl.BlockSpec((1, BR, CHN), lambda i, j: (j, i, 0)),
                  pl.BlockSpec((BR, CHN), lambda i, j: (i, 0)),
                  pl.BlockSpec((1, 16), lambda i, j: (0, 0)),
                  pl.BlockSpec((1, 16), lambda i, j: (0, 0)),
                  pl.BlockSpec((16, CHN), lambda i, j: (0, 0)),
                  pl.BlockSpec((1, CHN), lambda i, j: (0, 0)),
                  pl.BlockSpec((1, CHN), lambda i, j: (0, 0)),
                  pl.BlockSpec((1, CHN), lambda i, j: (0, 0)),
                  pl.BlockSpec((CHN, NSH), lambda i, j: (0, 0)),
                  pl.BlockSpec((1, NSH), lambda i, j: (0, 0))],
        out_specs=[pl.BlockSpec((1, BR, NSH), lambda i, j: (j, i, 0)),
                   pl.BlockSpec((8, NSH), lambda i, j: (0, 0))],
        out_shape=[jax.ShapeDtypeStruct((NS, NPTS, NSH), jnp.float32),
                   jax.ShapeDtypeStruct((8, NSH), jnp.float32)],
    )(y1a, kg3, xq, g1, bn1, w2p, b2, g2, bn2, ww1T, wb1)


def _fin_body(w1_ref, vg_ref, y1_ref, g1_ref, bn1_ref, w2p_ref, b2_ref,
              g3_ref, bn3_ref, ww2_ref, wb2_ref, out_ref):
    g3 = g3_ref[...]
    bn3 = bn3_ref[...]
    ww2 = ww2_ref[...]
    wb2 = wb2_ref[...]
    zs = []
    for j in range(NS):
        u = jax.nn.relu(w1_ref[j] * g3 + bn3)
        zs.append(jnp.dot(u, ww2, preferred_element_type=jnp.float32) + wb2)
    m = zs[0]
    for j in range(1, NS):
        m = jnp.maximum(m, zs[j])
    es = [jnp.exp(z - m) for z in zs]
    tot = es[0]
    for j in range(1, NS):
        tot = tot + es[j]
    inv = 1.0 / tot
    acc = jnp.zeros_like(out_ref)
    for j in range(NS):
        y = jax.nn.relu(y1_ref[j] * g1_ref[...] + bn1_ref[...])
        pr2 = jnp.dot(y, w2p_ref[...], preferred_element_type=jnp.float32) + b2_ref[...]
        v = vg_ref[j] + pr2
        wj = es[j] * inv
        wt = jnp.concatenate([wj] * 8, axis=1)
        acc = acc + v * wt
    out_ref[...] = acc


BRF = 400


def _final(w1a, vg3, y1a, g1, bn1, w2p, b2, g3, bn3, ww2T, wb2):
    grid = (NPTS // BRF,)
    return pl.pallas_call(
        _fin_body,
        grid=grid,
        in_specs=[pl.BlockSpec((NS, BRF, NSH), lambda i: (0, i, 0)),
                  pl.BlockSpec((NS, BRF, CHN), lambda i: (0, i, 0)),
                  pl.BlockSpec((NS, BRF, 16), lambda i: (0, i, 0)),
                  pl.BlockSpec((1, 16), lambda i: (0, 0)),
                  pl.BlockSpec((1, 16), lambda i: (0, 0)),
                  pl.BlockSpec((16, CHN), lambda i: (0, 0)),
                  pl.BlockSpec((1, CHN), lambda i: (0, 0)),
                  pl.BlockSpec((1, NSH), lambda i: (0, 0)),
                  pl.BlockSpec((1, NSH), lambda i: (0, 0)),
                  pl.BlockSpec((NSH, NSH), lambda i: (0, 0)),
                  pl.BlockSpec((1, NSH), lambda i: (0, 0))],
        out_specs=pl.BlockSpec((BRF, CHN), lambda i: (i, 0)),
        out_shape=jax.ShapeDtypeStruct((NPTS, CHN), jnp.float32),
    )(w1a, vg3, y1a, g1, bn1, w2p, b2, g3, bn3, ww2T, wb2)


def kernel(p, x, o, qw, qb, kw, kb, vw, vb, pw1, pb1, pg, pbeta, pw2, pb2,
           wg1, wbeta1, ww1, wlb1, wg2, wbeta2, ww2, wlb2):
    eps = 1e-5
    xq, xk, xv = _proj(x, qw, qb, kw, kb, vw, vb)

    pp = jnp.zeros((NPAD, 8), jnp.float32).at[:NPTS, :3].set(p)
    idx = _knn(pp, pp.T)[:NPTS]

    idx_t = idx.T.reshape(-1)
    pq16 = jnp.zeros((NPTS, CHN), jnp.float32).at[:, :3].set(p)
    kg, vg, pg_rows = _gather3(xk, xv, pq16, idx_t)
    kg3 = kg.reshape(NS, NPTS, CHN)
    vg3 = vg.reshape(NS, NPTS, CHN)
    pg3 = pg_rows.reshape(NS, NPTS, CHN)

    w1p = jnp.zeros((CHN, 16), jnp.float32).at[:3, :3].set(pw1.T)
    b1p = jnp.zeros((1, 16), jnp.float32).at[0, :3].set(pb1)
    w2p = jnp.zeros((16, CHN), jnp.float32).at[:3, :].set(pw2.T)
    b2 = pb2[None]
    pg_p = jnp.zeros((16,), jnp.float32).at[:3].set(pg)
    pbeta_p = jnp.zeros((16,), jnp.float32).at[:3].set(pbeta)

    y1a, s1 = _s1(pg3, pq16, w1p, b1p)
    m1 = s1[0] / MTOT
    v1 = s1[1] / MTOT - m1 * m1
    g1v = pg_p / jnp.sqrt(v1 + eps)
    g1 = g1v[None]
    bn1 = (pbeta_p - m1 * g1v)[None]

    s2 = _s2(y1a, kg3, xq, g1, bn1, w2p, b2)
    m2 = s2[0] / MTOT
    v2 = s2[1] / MTOT - m2 * m2
    g2v = wg1 / jnp.sqrt(v2 + eps)
    g2 = g2v[None]
    bn2 = (wbeta1 - m2 * g2v)[None]

    w1a, s3 = _w1(y1a, kg3, xq, g1, bn1, w2p, b2, g2, bn2,
                  ww1.T, wlb1[None])
    m3 = s3[0] / MTOT
    v3 = s3[1] / MTOT - m3 * m3
    g3v = wg2 / jnp.sqrt(v3 + eps)
    g3 = g3v[None]
    bn3 = (wbeta2 - m3 * g3v)[None]

    return _final(w1a, vg3, y1a, g1, bn1, w2p, b2,
                  g3, bn3, ww2.T, wlb2[None])
